# Initial kernel scaffold; baseline (speedup 1.0000x reference)
import functools

import jax
import jax.numpy as jnp
from jax import lax
from jax.experimental import pallas as pl
from jax.experimental.pallas import tpu as pltpu
from jax.experimental.pallas import tpu_sc as plsc

_B = 16384
_V = 53
_D = 80
_NC = 2
_NS = 16
_NW = _NC * _NS
_BPW = _B // _NW
_CH = 128
_NCH = _BPW // _CH


def _table_body(cat_ref, pw_ref, pb_ref, hw_ref, hb_ref, out_ref):
    card = jnp.dot(cat_ref[...], pw_ref[...],
                   preferred_element_type=jnp.float32) + pb_ref[...]
    out_ref[...] = jnp.dot(card, hw_ref[...],
                           preferred_element_type=jnp.float32) + hb_ref[...]


_table_call = pl.pallas_call(
    _table_body,
    out_shape=jax.ShapeDtypeStruct((_V, _D), jnp.float32),
)


@functools.partial(
    pl.kernel,
    mesh=plsc.VectorSubcoreMesh(core_axis_name="c", subcore_axis_name="s"),
    out_type=jax.ShapeDtypeStruct((_B, _D), jnp.float32),
    scratch_types=[
        pltpu.VMEM((_NCH, _CH), jnp.int32),
        pltpu.VMEM((_BPW, _D), jnp.float32),
        pltpu.SemaphoreType.DMA,
    ],
)
def _gather(table_hbm, idx_hbm, out_hbm, idx_v, rows_v, sem):
    wid = lax.axis_index("s") * _NC + lax.axis_index("c")
    pltpu.sync_copy(idx_hbm.at[pl.ds(wid * _NCH, _NCH)], idx_v)
    copies = [
        pltpu.async_copy(table_hbm.at[idx_v.at[j]],
                         rows_v.at[pl.ds(j * _CH, _CH)], sem)
        for j in range(_NCH)
    ]
    for c in copies:
        c.wait()
    pltpu.sync_copy(rows_v, out_hbm.at[pl.ds(wid * _BPW, _BPW)])


def kernel(card_id, rank_tab, suit_tab, dist_tab, proj_W, proj_b,
           rank_W, rank_b, suit_W, suit_b, dist_W, dist_b):
    cat_tab = jnp.concatenate([rank_tab, suit_tab, dist_tab], axis=1)
    heads_W = jnp.zeros((16, _D), jnp.float32)
    heads_W = heads_W.at[:, 0:13].set(rank_W)
    heads_W = heads_W.at[:, 13:17].set(suit_W)
    heads_W = heads_W.at[:, 17:69].set(dist_W)
    heads_b = jnp.zeros((_D,), jnp.float32)
    heads_b = heads_b.at[0:13].set(rank_b)
    heads_b = heads_b.at[13:17].set(suit_b)
    heads_b = heads_b.at[17:69].set(dist_b)

    table = _table_call(cat_tab, proj_W, proj_b.reshape(1, 16),
                        heads_W, heads_b.reshape(1, _D))
    idx = card_id.astype(jnp.int32).reshape(_B // _CH, _CH)
    out = _gather(table, idx)
    return out[:, 0:13], out[:, 13:17], out[:, 17:69]


# speedup vs baseline: 1.9043x; 1.9043x over previous
import functools

import jax
import jax.numpy as jnp
from jax import lax
from jax.experimental import pallas as pl
from jax.experimental.pallas import tpu as pltpu
from jax.experimental.pallas import tpu_sc as plsc

_B = 16384
_V = 53
_D = 80
_NC = 2
_NS = 16
_NW = _NC * _NS
_BPW = _B // _NW
_CH = 128
_NCH = _BPW // _CH


def _table_body(cat_ref, pw_ref, pb_ref, hw_ref, hb_ref, out_ref):
    card = jnp.dot(cat_ref[...], pw_ref[...],
                   preferred_element_type=jnp.float32) + pb_ref[...]
    out_ref[...] = jnp.dot(card, hw_ref[...],
                           preferred_element_type=jnp.float32) + hb_ref[...]


_table_call = pl.pallas_call(
    _table_body,
    out_shape=jax.ShapeDtypeStruct((_V, _D), jnp.float32),
)


@functools.cache
def _make_gather():
    @functools.partial(
        pl.kernel,
        mesh=plsc.VectorSubcoreMesh(core_axis_name="c", subcore_axis_name="s"),
        out_type=jax.ShapeDtypeStruct((_B, _D), jnp.float32),
        scratch_types=[
            pltpu.VMEM((_NCH, _CH), jnp.int32),
            pltpu.VMEM((_BPW, _D), jnp.float32),
            pltpu.SemaphoreType.DMA,
        ],
        compiler_params=pltpu.CompilerParams(use_tc_tiling_on_sc=False),
    )
    def _gather(table_hbm, idx_hbm, out_hbm, idx_v, rows_v, sem):
        wid = lax.axis_index("s") * _NC + lax.axis_index("c")
        pltpu.sync_copy(idx_hbm.at[pl.ds(wid * _NCH, _NCH)], idx_v)
        copies = [
            pltpu.async_copy(table_hbm.at[idx_v.at[j]],
                             rows_v.at[pl.ds(j * _CH, _CH)], sem)
            for j in range(_NCH)
        ]
        for c in copies:
            c.wait()
        pltpu.sync_copy(rows_v, out_hbm.at[pl.ds(wid * _BPW, _BPW)])

    return _gather


def kernel(card_id, rank_tab, suit_tab, dist_tab, proj_W, proj_b,
           rank_W, rank_b, suit_W, suit_b, dist_W, dist_b):
    cat_tab = jnp.concatenate([rank_tab, suit_tab, dist_tab], axis=1)
    heads_W = jnp.zeros((16, _D), jnp.float32)
    heads_W = heads_W.at[:, 0:13].set(rank_W)
    heads_W = heads_W.at[:, 13:17].set(suit_W)
    heads_W = heads_W.at[:, 17:69].set(dist_W)
    heads_b = jnp.zeros((_D,), jnp.float32)
    heads_b = heads_b.at[0:13].set(rank_b)
    heads_b = heads_b.at[13:17].set(suit_b)
    heads_b = heads_b.at[17:69].set(dist_b)

    table = _table_call(cat_tab, proj_W, proj_b.reshape(1, 16),
                        heads_W, heads_b.reshape(1, _D))
    idx = card_id.astype(jnp.int32).reshape(_B // _CH, _CH)
    out = _make_gather()(table, idx)
    return out[:, 0:13], out[:, 13:17], out[:, 17:69]


# speedup vs baseline: 2.0681x; 1.0860x over previous
import functools

import jax
import jax.numpy as jnp
from jax import lax
from jax.experimental import pallas as pl
from jax.experimental.pallas import tpu as pltpu
from jax.experimental.pallas import tpu_sc as plsc

_B = 16384
_V = 53
_D = 128
_NC = 2
_NS = 16
_NW = _NC * _NS
_BPW = _B // _NW
_CH = 128
_NCH = _BPW // _CH
_SPLIT_BLK = 2048


def _table_body(rank_ref, suit_ref, dist_ref, pw_ref, pb_ref,
                rw_ref, rb_ref, sw_ref, sb_ref, dw_ref, db_ref, out_ref):
    cat = jnp.concatenate([rank_ref[...], suit_ref[...], dist_ref[...]],
                          axis=1)
    card = jnp.dot(cat, pw_ref[...],
                   preferred_element_type=jnp.float32) + pb_ref[...]
    out_ref[...] = jnp.zeros((_V, _D), jnp.float32)
    out_ref[:, 0:13] = jnp.dot(card, rw_ref[...],
                               preferred_element_type=jnp.float32) + rb_ref[...]
    out_ref[:, 13:17] = jnp.dot(card, sw_ref[...],
                                preferred_element_type=jnp.float32) + sb_ref[...]
    out_ref[:, 17:69] = jnp.dot(card, dw_ref[...],
                                preferred_element_type=jnp.float32) + db_ref[...]


_table_call = pl.pallas_call(
    _table_body,
    out_shape=jax.ShapeDtypeStruct((_V, _D), jnp.float32),
)


@functools.cache
def _make_gather():
    @functools.partial(
        pl.kernel,
        mesh=plsc.VectorSubcoreMesh(core_axis_name="c", subcore_axis_name="s"),
        out_type=jax.ShapeDtypeStruct((_B, _D), jnp.float32),
        scratch_types=[
            pltpu.VMEM((_BPW,), jnp.int32),
            pltpu.VMEM((_BPW, _D), jnp.float32),
            pltpu.SemaphoreType.DMA,
        ],
    )
    def _gather(table_hbm, idx_hbm, out_hbm, idx_v, rows_v, sem):
        wid = lax.axis_index("s") * _NC + lax.axis_index("c")
        base = wid * _BPW
        pltpu.sync_copy(idx_hbm.at[pl.ds(base, _BPW)], idx_v)
        copies = [
            pltpu.async_copy(table_hbm.at[idx_v.at[pl.ds(j * _CH, _CH)]],
                             rows_v.at[pl.ds(j * _CH, _CH)], sem)
            for j in range(_NCH)
        ]
        for c in copies:
            c.wait()
        pltpu.sync_copy(rows_v, out_hbm.at[pl.ds(base, _BPW)])

    return _gather


def _split_body(in_ref, rank_ref, suit_ref, dist_ref):
    x = in_ref[...]
    rank_ref[...] = x[:, 0:13]
    suit_ref[...] = x[:, 13:17]
    dist_ref[...] = x[:, 17:69]


_split_call = pl.pallas_call(
    _split_body,
    grid=(_B // _SPLIT_BLK,),
    in_specs=[pl.BlockSpec((_SPLIT_BLK, _D), lambda i: (i, 0))],
    out_specs=[
        pl.BlockSpec((_SPLIT_BLK, 13), lambda i: (i, 0)),
        pl.BlockSpec((_SPLIT_BLK, 4), lambda i: (i, 0)),
        pl.BlockSpec((_SPLIT_BLK, 52), lambda i: (i, 0)),
    ],
    out_shape=[
        jax.ShapeDtypeStruct((_B, 13), jnp.float32),
        jax.ShapeDtypeStruct((_B, 4), jnp.float32),
        jax.ShapeDtypeStruct((_B, 52), jnp.float32),
    ],
)


def kernel(card_id, rank_tab, suit_tab, dist_tab, proj_W, proj_b,
           rank_W, rank_b, suit_W, suit_b, dist_W, dist_b):
    table = _table_call(rank_tab, suit_tab, dist_tab,
                        proj_W, proj_b.reshape(1, 16),
                        rank_W, rank_b.reshape(1, 13),
                        suit_W, suit_b.reshape(1, 4),
                        dist_W, dist_b.reshape(1, 52))
    idx = card_id.astype(jnp.int32)
    mid = _make_gather()(table, idx)
    rank_pred, suit_pred, dist_pred = _split_call(mid)
    return rank_pred, suit_pred, dist_pred
